# baseline (speedup 1.0000x reference)
import functools

import jax
import jax.numpy as jnp
from jax import lax
from jax.experimental import pallas as pl
from jax.experimental.pallas import tpu as pltpu
from jax.experimental.pallas import tpu_sc as plsc

B = 32
C = 256
A = 1024
T = 128
K = 7
CK = C * K
RPA = 16
ROW_W = (C * T) // RPA
NROWS = B * RPA


def _conv_argmax(x, wf, b2):
    def body(x_ref, w_ref, b_ref, idx_ref, rows_ref):
        xblk = x_ref[0]
        lane = lax.broadcasted_iota(jnp.int32, (C, T), 1)
        parts = [xblk]
        for k in range(1, K):
            sh = pltpu.roll(xblk, T - k, axis=1)
            sh = jnp.where(lane < T - k, sh, jnp.float32(0))
            parts.append(sh)
        xstack = jnp.concatenate(parts, axis=0)
        acc = jnp.dot(w_ref[...], xstack,
                      preferred_element_type=jnp.float32,
                      precision=lax.Precision.HIGHEST)
        val = acc + b_ref[...]
        m = jnp.max(val)
        flat = (lax.broadcasted_iota(jnp.int32, (A, T), 0) * T
                + lax.broadcasted_iota(jnp.int32, (A, T), 1))
        cand = jnp.where(val == m, flat, jnp.int32(2147483647))
        i = pl.program_id(0)
        best = jnp.min(cand)
        idx_ref[i] = best
        ai = best >> 7
        for j in range(RPA):
            rows_ref[i * RPA + j] = ai * RPA + j

    return pl.pallas_call(
        body,
        grid=(B,),
        in_specs=[
            pl.BlockSpec((1, C, T), lambda i: (i, 0, 0)),
            pl.BlockSpec((A, CK), lambda i: (0, 0)),
            pl.BlockSpec((A, 1), lambda i: (0, 0)),
        ],
        out_specs=[
            pl.BlockSpec((B,), lambda i: (0,), memory_space=pltpu.SMEM),
            pl.BlockSpec((NROWS,), lambda i: (0,), memory_space=pltpu.SMEM),
        ],
        out_shape=[
            jax.ShapeDtypeStruct((B,), jnp.int32),
            jax.ShapeDtypeStruct((NROWS,), jnp.int32),
        ],
    )(x, wf, b2)


def _sc_gather(rows, atoms2d):
    info = plsc.get_sparse_core_info()
    nc, ns = info.num_cores, info.num_subcores
    nw = nc * ns
    rpw = NROWS // nw
    mesh = plsc.VectorSubcoreMesh(core_axis_name="c", subcore_axis_name="s")

    @functools.partial(
        pl.kernel,
        mesh=mesh,
        out_type=jax.ShapeDtypeStruct((NROWS, ROW_W), jnp.float32),
        scratch_types=[
            pltpu.VMEM((rpw,), jnp.int32),
            pltpu.VMEM((rpw, ROW_W), jnp.float32),
            pltpu.SemaphoreType.DMA,
        ],
    )
    def k(rows_hbm, atoms_hbm, out_hbm, idx_v, rows_v, sem):
        wid = lax.axis_index("s") * nc + lax.axis_index("c")
        base = wid * rpw
        pltpu.sync_copy(rows_hbm.at[pl.ds(base, rpw)], idx_v)
        pltpu.async_copy(atoms_hbm.at[idx_v], rows_v, sem).wait()
        pltpu.sync_copy(rows_v, out_hbm.at[pl.ds(base, rpw)])

    return k(rows, atoms2d)


def _finish(idx, x, praw):
    def body(idx_ref, x_ref, p_ref, sc_ref, re_ref):
        i = pl.program_id(0)
        pos = idx_ref[i] & (T - 1)
        p0 = p_ref[0]
        u = lax.broadcasted_iota(jnp.int32, (T, T), 0)
        t = lax.broadcasted_iota(jnp.int32, (T, T), 1)
        shift = jnp.where(u + pos == t, 1.0, 0.0).astype(jnp.float32)
        p = lax.dot(p0, shift, precision=lax.Precision.HIGHEST)
        n = jnp.sqrt(jnp.sum(p * p))
        normed = p / (n + 1e-8)
        xblk = x_ref[0]
        corr = jnp.sum(xblk * normed)
        scaled = normed * corr
        sc_ref[0] = scaled
        re_ref[0] = xblk - scaled

    return pl.pallas_call(
        body,
        grid=(B,),
        in_specs=[
            pl.BlockSpec(memory_space=pltpu.SMEM),
            pl.BlockSpec((1, C, T), lambda i: (i, 0, 0)),
            pl.BlockSpec((1, C, T), lambda i: (i, 0, 0)),
        ],
        out_specs=[
            pl.BlockSpec((1, C, T), lambda i: (i, 0, 0)),
            pl.BlockSpec((1, C, T), lambda i: (i, 0, 0)),
        ],
        out_shape=[
            jax.ShapeDtypeStruct((B, C, T), jnp.float32),
            jax.ShapeDtypeStruct((B, C, T), jnp.float32),
        ],
    )(idx, x, praw)


def kernel(x, W, b, atoms):
    wf = W.transpose(0, 2, 1).reshape(A, CK)
    b2 = b.reshape(A, 1)
    idx, rows = _conv_argmax(x, wf, b2)
    praw = _sc_gather(rows, atoms.reshape(A * RPA, ROW_W)).reshape(B, C, T)
    scaled, residual = _finish(idx, x, praw)
    return (scaled, residual)
